# baseline (speedup 1.0000x reference)
import jax
import jax.numpy as jnp
from jax import lax
from jax.experimental import pallas as pl
from jax.experimental.pallas import tpu as pltpu
from jax.experimental.pallas import tpu_sc as plsc

NC = 2
NS = 16
NW = NC * NS
D = 64
CHUNK = 128
GRP = 16


def _body(value, depth, position, src_tbl, tdp, sp1, sp2, out,
          iv, idep, posv, tdp_v, sp1_v, sp2_v, acc, sem):
    wid = lax.axis_index("s") * NC + lax.axis_index("c")
    n_chunks = iv.shape[0]

    pltpu.sync_copy(value.at[wid], iv)
    pltpu.sync_copy(depth.at[wid], idep)
    pltpu.sync_copy(position.at[wid], posv)
    pltpu.sync_copy(tdp, tdp_v)
    pltpu.sync_copy(sp1, sp1_v)
    pltpu.sync_copy(sp2, sp2_v)

    cols = jnp.arange(GRP, dtype=jnp.int32)
    c0 = jnp.zeros((GRP,), jnp.int32)
    descs = []
    for k in range(n_chunks):
        @pl.loop(0, CHUNK // GRP)
        def _grp(g, k=k):
            base_t = k * CHUNK + g * GRP
            tvec = cols + base_t
            dvec = idep[pl.ds(base_t, GRP)]
            p0 = plsc.load_gather(posv, [tvec, c0])
            p1 = plsc.load_gather(posv, [tvec, c0 + 1])
            p2 = plsc.load_gather(posv, [tvec, c0 + 2])
            rdp = dvec * 64 + p0
            for l in range(GRP):
                sel = jnp.full((GRP,), l, jnp.int32)
                r_dp = rdp.at[sel].get(mode="promise_in_bounds")
                r_p1 = p1.at[sel].get(mode="promise_in_bounds")
                r_p2 = p2.at[sel].get(mode="promise_in_bounds")
                for c in range(D // GRP):
                    colv = cols + (GRP * c)
                    x = (plsc.load_gather(tdp_v, [r_dp, colv])
                         + plsc.load_gather(sp1_v, [r_p1, colv])) \
                        + plsc.load_gather(sp2_v, [r_p2, colv])
                    acc[base_t + l, pl.ds(GRP * c, GRP)] = x

        dst = acc.at[pl.ds(k * CHUNK, CHUNK)]
        descs.append(
            pltpu.async_copy(src_tbl.at[iv.at[k]], dst, sem, add=True))

    for d in descs:
        d.wait()

    pltpu.sync_copy(acc, out.at[pl.ds(wid * n_chunks * CHUNK,
                                      n_chunks * CHUNK)])


@jax.jit
def _run(value, depth, position, src_table, tdp, sp1, sp2):
    n_tok = value.size
    n_chunks = n_tok // (NW * CHUNK)
    t_per_w = n_chunks * CHUNK
    mesh = plsc.VectorSubcoreMesh(core_axis_name="c", subcore_axis_name="s")
    f = pl.kernel(
        _body,
        out_type=jax.ShapeDtypeStruct((n_tok, D), jnp.float32),
        mesh=mesh,
        scratch_types=[
            pltpu.VMEM((n_chunks, CHUNK), jnp.int32),
            pltpu.VMEM((t_per_w,), jnp.int32),
            pltpu.VMEM((t_per_w, 3), jnp.int32),
            pltpu.VMEM(tdp.shape, jnp.float32),
            pltpu.VMEM(sp1.shape, jnp.float32),
            pltpu.VMEM(sp2.shape, jnp.float32),
            pltpu.VMEM((t_per_w, D), jnp.float32),
            pltpu.SemaphoreType.DMA,
        ],
        compiler_params=pltpu.CompilerParams(use_tc_tiling_on_sc=False,
                                             needs_layout_passes=False),
    )
    return f(value.reshape(NW, n_chunks, CHUNK),
             depth.reshape(NW, t_per_w),
             position.reshape(NW, t_per_w, 3),
             src_table, tdp, sp1, sp2)


def kernel(value, depth, position, src_table, depth_table, spatial_tables):
    B, S = value.shape
    tdp = (depth_table[:, None, :] + spatial_tables[0][None, :, :]).reshape(
        -1, D)
    out = _run(value, depth, position, src_table, tdp,
               spatial_tables[1], spatial_tables[2])
    return out.reshape(B, S, D)


# speedup vs baseline: 1.2452x; 1.2452x over previous
import jax
import jax.numpy as jnp
from jax import lax
from jax.experimental import pallas as pl
from jax.experimental.pallas import tpu as pltpu
from jax.experimental.pallas import tpu_sc as plsc

NC = 2
NS = 16
NW = NC * NS
D = 64
CHUNK = 128
GRP = 16


def _body(value, dp, p1, p2, src_tbl, tdp, sp1, sp2, out,
          iv, idp, i1, i2, tdp_v, sp1_v, sp2_v, acc, sem):
    wid = lax.axis_index("s") * NC + lax.axis_index("c")
    n_chunks = iv.shape[0]
    t_per_w = n_chunks * CHUNK
    s_per_b = out.shape[1]
    w_per_b = s_per_b // t_per_w
    b = wid // w_per_b
    s0 = (wid % w_per_b) * t_per_w

    pltpu.sync_copy(value.at[wid], iv)
    pltpu.sync_copy(dp.at[wid], idp)
    pltpu.sync_copy(p1.at[wid], i1)
    pltpu.sync_copy(p2.at[wid], i2)
    pltpu.sync_copy(tdp, tdp_v)
    pltpu.sync_copy(sp1, sp1_v)
    pltpu.sync_copy(sp2, sp2_v)

    cols = jnp.arange(GRP, dtype=jnp.int32)

    @pl.loop(0, n_chunks)
    def _chunk(k):
        @pl.loop(0, CHUNK // GRP)
        def _grp(g):
            base_t = k * CHUNK + g * GRP
            dpv = idp[k, pl.ds(g * GRP, GRP)]
            p1v = i1[k, pl.ds(g * GRP, GRP)]
            p2v = i2[k, pl.ds(g * GRP, GRP)]
            for l in range(GRP):
                sel = jnp.full((GRP,), l, jnp.int32)
                r_dp = dpv.at[sel].get(mode="promise_in_bounds")
                r_p1 = p1v.at[sel].get(mode="promise_in_bounds")
                r_p2 = p2v.at[sel].get(mode="promise_in_bounds")
                for c in range(D // GRP):
                    colv = cols + (GRP * c)
                    x = (plsc.load_gather(tdp_v, [r_dp, colv])
                         + plsc.load_gather(sp1_v, [r_p1, colv])) \
                        + plsc.load_gather(sp2_v, [r_p2, colv])
                    acc[base_t + l, pl.ds(GRP * c, GRP)] = x

        pltpu.async_copy(src_tbl.at[iv.at[k]],
                         acc.at[pl.ds(k * CHUNK, CHUNK)], sem, add=True)

    pltpu.make_async_copy(out.at[b, pl.ds(s0, t_per_w)], acc, sem).wait()
    pltpu.sync_copy(acc, out.at[b, pl.ds(s0, t_per_w)])


@jax.jit
def _run(value, dp, p1, p2, src_table, tdp, sp1, sp2):
    B, S = value.shape
    n_tok = value.size
    n_chunks = n_tok // (NW * CHUNK)
    shape3 = (NW, n_chunks, CHUNK)
    mesh = plsc.VectorSubcoreMesh(core_axis_name="c", subcore_axis_name="s")
    f = pl.kernel(
        _body,
        out_type=jax.ShapeDtypeStruct((B, S, D), jnp.float32),
        mesh=mesh,
        scratch_types=[
            pltpu.VMEM((n_chunks, CHUNK), jnp.int32),
            pltpu.VMEM((n_chunks, CHUNK), jnp.int32),
            pltpu.VMEM((n_chunks, CHUNK), jnp.int32),
            pltpu.VMEM((n_chunks, CHUNK), jnp.int32),
            pltpu.VMEM(tdp.shape, jnp.float32),
            pltpu.VMEM(sp1.shape, jnp.float32),
            pltpu.VMEM(sp2.shape, jnp.float32),
            pltpu.VMEM((n_chunks * CHUNK, D), jnp.float32),
            pltpu.SemaphoreType.DMA,
        ],
        compiler_params=pltpu.CompilerParams(use_tc_tiling_on_sc=False,
                                             needs_layout_passes=False),
    )
    return f(value.reshape(shape3), dp.reshape(shape3), p1.reshape(shape3),
             p2.reshape(shape3), src_table, tdp, sp1, sp2)


def kernel(value, depth, position, src_table, depth_table, spatial_tables):
    B, S = value.shape
    tdp = (depth_table[:, None, :] + spatial_tables[0][None, :, :]).reshape(
        -1, D)
    dp_idx = depth * 64 + position[:, :, 0]
    return _run(value, dp_idx, position[:, :, 1], position[:, :, 2],
                src_table, tdp, spatial_tables[1], spatial_tables[2])


# speedup vs baseline: 1.3595x; 1.0917x over previous
import jax
import jax.numpy as jnp
from jax import lax
from jax.experimental import pallas as pl
from jax.experimental.pallas import tpu as pltpu
from jax.experimental.pallas import tpu_sc as plsc

NC = 2
NS = 16
NW = NC * NS
D = 64
CHUNK = 128
GRP = 16


def _body(value, depth, pcols, src_tbl, tdp, sp1, sp2, out,
          iv, idp, i0, i1, i2, tdp_v, sp1_v, sp2_v, acc, sem):
    wid = lax.axis_index("s") * NC + lax.axis_index("c")
    n_chunks = iv.shape[0]
    t_per_w = n_chunks * CHUNK
    s_per_b = out.shape[1]
    w_per_b = s_per_b // t_per_w
    b = wid // w_per_b
    s0 = (wid % w_per_b) * t_per_w

    pltpu.sync_copy(value.at[wid], iv)
    pltpu.sync_copy(depth.at[wid], idp)
    pltpu.sync_copy(pcols.at[0, wid], i0)
    pltpu.sync_copy(pcols.at[1, wid], i1)
    pltpu.sync_copy(pcols.at[2, wid], i2)
    pltpu.sync_copy(tdp, tdp_v)
    pltpu.sync_copy(sp1, sp1_v)
    pltpu.sync_copy(sp2, sp2_v)

    cols = jnp.arange(GRP, dtype=jnp.int32)

    @pl.loop(0, n_chunks)
    def _chunk(k):
        @pl.loop(0, CHUNK // GRP)
        def _grp(g):
            base_t = k * CHUNK + g * GRP
            dvec = idp[k, pl.ds(g * GRP, GRP)]
            p0v = i0[k, pl.ds(g * GRP, GRP)]
            dpv = dvec * 64 + p0v
            p1v = i1[k, pl.ds(g * GRP, GRP)]
            p2v = i2[k, pl.ds(g * GRP, GRP)]
            for l in range(GRP):
                sel = jnp.full((GRP,), l, jnp.int32)
                r_dp = dpv.at[sel].get(mode="promise_in_bounds")
                r_p1 = p1v.at[sel].get(mode="promise_in_bounds")
                r_p2 = p2v.at[sel].get(mode="promise_in_bounds")
                for c in range(D // GRP):
                    colv = cols + (GRP * c)
                    x = (plsc.load_gather(tdp_v, [r_dp, colv])
                         + plsc.load_gather(sp1_v, [r_p1, colv])) \
                        + plsc.load_gather(sp2_v, [r_p2, colv])
                    acc[base_t + l, pl.ds(GRP * c, GRP)] = x

        pltpu.async_copy(src_tbl.at[iv.at[k]],
                         acc.at[pl.ds(k * CHUNK, CHUNK)], sem, add=True)

    dst = out.at[b, pl.ds(s0, t_per_w), pl.ds(0, D)]
    pltpu.make_async_copy(dst, acc, sem).wait()
    pltpu.sync_copy(acc, dst)


@jax.jit
def _run(value, depth, pcols, src_table, tdp, sp1, sp2):
    B, S = value.shape
    n_tok = value.size
    n_chunks = n_tok // (NW * CHUNK)
    shape3 = (NW, n_chunks, CHUNK)
    mesh = plsc.VectorSubcoreMesh(core_axis_name="c", subcore_axis_name="s")
    f = pl.kernel(
        _body,
        out_type=jax.ShapeDtypeStruct((B, S, 128), jnp.float32),
        mesh=mesh,
        scratch_types=[
            pltpu.VMEM((n_chunks, CHUNK), jnp.int32),
            pltpu.VMEM((n_chunks, CHUNK), jnp.int32),
            pltpu.VMEM((n_chunks, CHUNK), jnp.int32),
            pltpu.VMEM((n_chunks, CHUNK), jnp.int32),
            pltpu.VMEM((n_chunks, CHUNK), jnp.int32),
            pltpu.VMEM(tdp.shape, jnp.float32),
            pltpu.VMEM(sp1.shape, jnp.float32),
            pltpu.VMEM(sp2.shape, jnp.float32),
            pltpu.VMEM((n_chunks * CHUNK, D), jnp.float32),
            pltpu.SemaphoreType.DMA,
        ],
        compiler_params=pltpu.CompilerParams(use_tc_tiling_on_sc=False,
                                             needs_layout_passes=False),
    )
    return f(value.reshape(shape3), depth.reshape(shape3),
             pcols.reshape((3,) + shape3), src_table, tdp, sp1, sp2)


def kernel(value, depth, position, src_table, depth_table, spatial_tables):
    tdp = (depth_table[:, None, :] + spatial_tables[0][None, :, :]).reshape(
        -1, D)
    pcols = jnp.moveaxis(position, 2, 0)
    out = _run(value, depth, pcols, src_table, tdp,
               spatial_tables[1], spatial_tables[2])
    return out[:, :, :D]


# speedup vs baseline: 1.5431x; 1.1350x over previous
import jax
import jax.numpy as jnp
from jax import lax
from jax.experimental import pallas as pl
from jax.experimental.pallas import tpu as pltpu
from jax.experimental.pallas import tpu_sc as plsc

NC = 2
NS = 16
NW = NC * NS
D = 64
CHUNK = 128
GRP = 16


def _body(value, depth, pcols, src_tbl, tdp, sp1, sp2, out,
          iv, idp, i0, i1, i2, tdp_v, sp1_v, sp2_v, acc, sem):
    wid = lax.axis_index("s") * NC + lax.axis_index("c")
    n_chunks = iv.shape[0]
    t_per_w = n_chunks * CHUNK
    s_per_b = out.shape[1]
    w_per_b = s_per_b // t_per_w
    b = wid // w_per_b
    s0 = (wid % w_per_b) * t_per_w

    pltpu.sync_copy(value.at[wid], iv)
    pltpu.sync_copy(depth.at[wid], idp)
    pltpu.sync_copy(pcols.at[0, wid], i0)
    pltpu.sync_copy(pcols.at[1, wid], i1)
    pltpu.sync_copy(pcols.at[2, wid], i2)
    pltpu.sync_copy(tdp, tdp_v)
    pltpu.sync_copy(sp1, sp1_v)
    pltpu.sync_copy(sp2, sp2_v)

    cols = jnp.arange(GRP, dtype=jnp.int32)

    @pl.loop(0, n_chunks)
    def _chunk(k):
        @plsc.parallel_loop(0, CHUNK // GRP)
        def _grp(g):
            base_t = k * CHUNK + g * GRP
            dvec = idp[k, pl.ds(g * GRP, GRP)]
            p0v = i0[k, pl.ds(g * GRP, GRP)]
            dpv = dvec * 64 + p0v
            p1v = i1[k, pl.ds(g * GRP, GRP)]
            p2v = i2[k, pl.ds(g * GRP, GRP)]
            for l in range(GRP):
                sel = jnp.full((GRP,), l, jnp.int32)
                r_dp = dpv.at[sel].get(mode="promise_in_bounds")
                r_p1 = p1v.at[sel].get(mode="promise_in_bounds")
                r_p2 = p2v.at[sel].get(mode="promise_in_bounds")
                for c in range(D // GRP):
                    colv = cols + (GRP * c)
                    x = (plsc.load_gather(tdp_v, [r_dp, colv])
                         + plsc.load_gather(sp1_v, [r_p1, colv])) \
                        + plsc.load_gather(sp2_v, [r_p2, colv])
                    acc[base_t + l, pl.ds(GRP * c, GRP)] = x

        pltpu.async_copy(src_tbl.at[iv.at[k]],
                         acc.at[pl.ds(k * CHUNK, CHUNK)], sem, add=True)

    dst = out.at[b, pl.ds(s0, t_per_w), pl.ds(0, D)]
    pltpu.make_async_copy(dst, acc, sem).wait()
    pltpu.sync_copy(acc, dst)


@jax.jit
def _run(value, depth, pcols, src_table, tdp, sp1, sp2):
    B, S = value.shape
    n_tok = value.size
    n_chunks = n_tok // (NW * CHUNK)
    shape3 = (NW, n_chunks, CHUNK)
    mesh = plsc.VectorSubcoreMesh(core_axis_name="c", subcore_axis_name="s")
    f = pl.kernel(
        _body,
        out_type=jax.ShapeDtypeStruct((B, S, 128), jnp.float32),
        mesh=mesh,
        scratch_types=[
            pltpu.VMEM((n_chunks, CHUNK), jnp.int32),
            pltpu.VMEM((n_chunks, CHUNK), jnp.int32),
            pltpu.VMEM((n_chunks, CHUNK), jnp.int32),
            pltpu.VMEM((n_chunks, CHUNK), jnp.int32),
            pltpu.VMEM((n_chunks, CHUNK), jnp.int32),
            pltpu.VMEM(tdp.shape, jnp.float32),
            pltpu.VMEM(sp1.shape, jnp.float32),
            pltpu.VMEM(sp2.shape, jnp.float32),
            pltpu.VMEM((n_chunks * CHUNK, D), jnp.float32),
            pltpu.SemaphoreType.DMA,
        ],
        compiler_params=pltpu.CompilerParams(use_tc_tiling_on_sc=False,
                                             needs_layout_passes=False),
    )
    return f(value.reshape(shape3), depth.reshape(shape3),
             pcols.reshape((3,) + shape3), src_table, tdp, sp1, sp2)


def kernel(value, depth, position, src_table, depth_table, spatial_tables):
    tdp = (depth_table[:, None, :] + spatial_tables[0][None, :, :]).reshape(
        -1, D)
    pcols = jnp.moveaxis(position, 2, 0)
    out = _run(value, depth, pcols, src_table, tdp,
               spatial_tables[1], spatial_tables[2])
    return out[:, :, :D]
